# baseline (speedup 1.0000x reference)
import dataclasses
import functools

import jax
import jax.numpy as jnp
from jax import lax
from jax.experimental import pallas as pl
from jax.experimental.pallas import tpu as pltpu
from jax.experimental.pallas import tpu_sc as plsc

_L, _B, _A = 2048, 1024, 21
_NW = 32
_CR, _CC = 8, 256
_NCHR = _L // _CR
_NCHC = _B // _CC
_NCH = _NCHR * _NCHC
_PER_W = _NCH // _NW
_LANES = 16


def _sc_onehot(aa_hbm, out_hbm, idx0, idx1, idx2, idx3, buf0, buf1,
               sem0, sem1, semi0, semi1, semi2, semi3):
    wid = lax.axis_index("s") * 2 + lax.axis_index("c")
    ones = jnp.full((_LANES,), 1.0, jnp.float32)
    zeros_f = jnp.zeros((_LANES,), jnp.float32)
    lane = lax.iota(jnp.int32, _LANES)
    idx_v = (idx0, idx1, idx2, idx3)
    buf_v = (buf0, buf1)
    sems = (sem0, sem1)
    sems_i = (semi0, semi1, semi2, semi3)

    def rowcol(k):
        m = wid * _PER_W + k
        return (m // _NCHC) * _CR, (m % _NCHC) * _CC

    def aa_slice(k):
        r0, c0 = rowcol(k)
        return aa_hbm.at[pl.ds(r0, _CR), pl.ds(c0, _CC)]

    def out_slice(k):
        r0, c0 = rowcol(k)
        return out_hbm.at[:, pl.ds(r0, _CR), pl.ds(c0, _CC)]

    def scatter_all(buf, idx, val):
        for r in range(_CR):
            rv = jnp.full((_LANES,), r, jnp.int32)

            @pl.loop(0, _CC, step=_LANES)
            def _(c):
                av = idx[r, pl.ds(c, _LANES)]
                plsc.store_scatter(buf, [av, rv, c + lane], val)

    for b in range(2):
        @pl.loop(0, _A)
        def _(a):
            @pl.loop(0, _CR)
            def _(r):
                @pl.loop(0, _CC, step=_LANES)
                def _(c):
                    buf_v[b][a, r, pl.ds(c, _LANES)] = zeros_f

    for p in range(2):
        pltpu.async_copy(aa_slice(p), idx_v[p], sems_i[p])

    @pl.loop(0, _PER_W // 4)
    def _(kk):
        for q in range(4):
            k = kk * 4 + q
            b = q % 2
            p = q
            pm2 = (q + 2) % 4

            @pl.when(k >= 2)
            def _():
                pltpu.make_async_copy(buf_v[b], out_slice(k), sems[b]).wait()
                scatter_all(buf_v[b], idx_v[pm2], zeros_f)

            pltpu.make_async_copy(aa_slice(k), idx_v[p], sems_i[p]).wait()
            scatter_all(buf_v[b], idx_v[p], ones)
            pltpu.async_copy(buf_v[b], out_slice(k), sems[b])

            @pl.when(k < _PER_W - 2)
            def _():
                pltpu.async_copy(aa_slice(k + 2), idx_v[pm2], sems_i[pm2])

    for b in range(2):
        pltpu.make_async_copy(
            buf_v[b], out_slice(_PER_W - 2 + b), sems[b]).wait()


def kernel(aa_indices, embed_tensor):
    del embed_tensor
    mesh = plsc.VectorSubcoreMesh(core_axis_name="c", subcore_axis_name="s")
    cp = pltpu.CompilerParams()
    if "needs_layout_passes" in pltpu.CompilerParams.__dataclass_fields__:
        cp = dataclasses.replace(cp, needs_layout_passes=False)
    sc_call = pl.kernel(
        _sc_onehot,
        out_type=jax.ShapeDtypeStruct((_A, _L, _B), jnp.float32),
        mesh=mesh,
        scratch_types=[
            pltpu.VMEM((_CR, _CC), jnp.int32),
            pltpu.VMEM((_CR, _CC), jnp.int32),
            pltpu.VMEM((_CR, _CC), jnp.int32),
            pltpu.VMEM((_CR, _CC), jnp.int32),
            pltpu.VMEM((_A, _CR, _CC), jnp.float32),
            pltpu.VMEM((_A, _CR, _CC), jnp.float32),
            pltpu.SemaphoreType.DMA,
            pltpu.SemaphoreType.DMA,
            pltpu.SemaphoreType.DMA,
            pltpu.SemaphoreType.DMA,
            pltpu.SemaphoreType.DMA,
            pltpu.SemaphoreType.DMA,
        ],
        compiler_params=cp,
    )
    return jnp.transpose(sc_call(aa_indices), (1, 2, 0))


# speedup vs baseline: 1.1474x; 1.1474x over previous
import dataclasses
import functools

import jax
import jax.numpy as jnp
from jax import lax
from jax.experimental import pallas as pl
from jax.experimental.pallas import tpu as pltpu
from jax.experimental.pallas import tpu_sc as plsc

_L, _B, _A = 2048, 1024, 21
_NW = 32
_CR, _CC = 8, 256
_NCHR = _L // _CR
_NCHC = _B // _CC
_NCH = _NCHR * _NCHC
_PER_W = _NCH // _NW
_LANES = 16


def _sc_onehot(aa_hbm, out_hbm, idx0, idx1, buf0, buf1,
               sem0, sem1, semi0, semi1):
    wid = lax.axis_index("s") * 2 + lax.axis_index("c")
    ones = jnp.full((_LANES,), 1.0, jnp.float32)
    zeros_f = jnp.zeros((_LANES,), jnp.float32)
    idx_v = (idx0, idx1)
    buf_v = (buf0, buf1)
    sems = (sem0, sem1)
    sems_i = (semi0, semi1)

    def rowcol(k):
        m = wid * _PER_W + k
        return (m // _NCHC) * _CR, (m % _NCHC) * _CC

    def aa_slice(k):
        r0, c0 = rowcol(k)
        return aa_hbm.at[pl.ds(r0, _CR), pl.ds(c0, _CC)]

    def out_slice(k):
        r0, c0 = rowcol(k)
        return out_hbm.at[:, pl.ds(r0, _CR), pl.ds(c0, _CC)]

    for b in range(2):
        pltpu.async_copy(aa_slice(b), idx_v[b], sems_i[b])

    @pl.loop(0, _PER_W // 2)
    def _(kk):
        for b in range(2):
            k = kk * 2 + b

            @pl.when(kk > 0)
            def _():
                pltpu.make_async_copy(buf_v[b], out_slice(k), sems[b]).wait()

            pltpu.make_async_copy(aa_slice(k), idx_v[b], sems_i[b]).wait()

            for r in range(_CR):

                @pl.loop(0, _CC, step=_LANES)
                def _(c):
                    av = idx_v[b][r, pl.ds(c, _LANES)]
                    for a in range(_A):
                        buf_v[b][a, r, pl.ds(c, _LANES)] = jnp.where(
                            av == a, ones, zeros_f)

            pltpu.async_copy(buf_v[b], out_slice(k), sems[b])

            @pl.when(kk < _PER_W // 2 - 1)
            def _():
                pltpu.async_copy(aa_slice(k + 2), idx_v[b], sems_i[b])

    for b in range(2):
        pltpu.make_async_copy(
            buf_v[b], out_slice(_PER_W - 2 + b), sems[b]).wait()


def kernel(aa_indices, embed_tensor):
    del embed_tensor
    mesh = plsc.VectorSubcoreMesh(core_axis_name="c", subcore_axis_name="s")
    cp = pltpu.CompilerParams()
    if "needs_layout_passes" in pltpu.CompilerParams.__dataclass_fields__:
        cp = dataclasses.replace(cp, needs_layout_passes=False)
    sc_call = pl.kernel(
        _sc_onehot,
        out_type=jax.ShapeDtypeStruct((_A, _L, _B), jnp.float32),
        mesh=mesh,
        scratch_types=[
            pltpu.VMEM((_CR, _CC), jnp.int32),
            pltpu.VMEM((_CR, _CC), jnp.int32),
            pltpu.VMEM((_A, _CR, _CC), jnp.float32),
            pltpu.VMEM((_A, _CR, _CC), jnp.float32),
            pltpu.SemaphoreType.DMA,
            pltpu.SemaphoreType.DMA,
            pltpu.SemaphoreType.DMA,
            pltpu.SemaphoreType.DMA,
        ],
        compiler_params=cp,
    )
    return jnp.transpose(sc_call(aa_indices), (1, 2, 0))
